# baseline (speedup 1.0000x reference)
import functools

import jax
import jax.numpy as jnp
from jax import lax
from jax.experimental import pallas as pl
from jax.experimental.pallas import tpu as pltpu
from jax.experimental.pallas import tpu_sc as plsc

N = 10000
E = 320000
D = 128
NC = 2
NS = 16
NW = NC * NS
SPLITS = (128000, 192000)
OFFS = (0, 128000)
CG = 80
CS = 40
NBUF = 5
STRIPE = 632
NP = NS * STRIPE

_mesh = plsc.VectorSubcoreMesh(core_axis_name="c", subcore_axis_name="s")


def _make_gather(eh):
    ewh = eh // NW
    nch = ewh // CG
    assert ewh % 8 == 0 and nch % NBUF == 0

    @functools.partial(
        pl.kernel,
        out_type=jax.ShapeDtypeStruct((eh, D), jnp.float32),
        mesh=_mesh,
        scratch_types=[
            pltpu.VMEM((nch, CG), jnp.int32),
            pltpu.VMEM((nch, CG), jnp.int32),
            pltpu.VMEM((NBUF, CG, D), jnp.float32),
            pltpu.SemaphoreType.DMA((NBUF,)),
            pltpu.SemaphoreType.DMA((NBUF,)),
            pltpu.SemaphoreType.DMA((NBUF,)),
        ],
    )
    def gather_sum(ps_hbm, pr_hbm, snd_hbm, rcv_hbm, out_hbm,
                   idx_s, idx_r, bufs, gsem, asem, wsem):
        w = lax.axis_index("s") * NC + lax.axis_index("c")
        base0 = w * ewh
        pltpu.sync_copy(snd_hbm.at[w], idx_s)
        pltpu.sync_copy(rcv_hbm.at[w], idx_r)

        def gs(j, b):
            pltpu.async_copy(ps_hbm.at[idx_s.at[j]], bufs.at[b], gsem.at[b])

        def ga(j, b):
            pltpu.make_async_copy(ps_hbm.at[idx_s.at[j]], bufs.at[b],
                                  gsem.at[b]).wait()
            pltpu.async_copy(pr_hbm.at[idx_r.at[j]], bufs.at[b], asem.at[b],
                             add=True)

        def wr(j, b):
            pltpu.make_async_copy(pr_hbm.at[idx_r.at[j]], bufs.at[b],
                                  asem.at[b]).wait()
            pltpu.async_copy(bufs.at[b],
                             out_hbm.at[pl.ds(base0 + j * CG, CG), :],
                             wsem.at[b])

        def wr_wait(j, b):
            pltpu.make_async_copy(bufs.at[b],
                                  out_hbm.at[pl.ds(base0 + j * CG, CG), :],
                                  wsem.at[b]).wait()

        def outer(i, carry):
            for b in range(NBUF):
                j = i * NBUF + b

                @pl.when(j >= 2)
                def _():
                    wr(j - 2, (b - 2) % NBUF)

                @pl.when(j >= 1)
                def _():
                    ga(j - 1, (b - 1) % NBUF)

                @pl.when(j >= NBUF)
                def _():
                    wr_wait(j - NBUF, b)

                gs(j, b)
            return carry

        lax.fori_loop(0, nch // NBUF, outer, 0)
        ga(nch - 1, (nch - 1) % NBUF)
        wr(nch - 2, (nch - 2) % NBUF)
        wr(nch - 1, (nch - 1) % NBUF)
        for b in range(NBUF):
            wr_wait(nch - NBUF + b, b)

    return gather_sum


def _make_seg_sum(eh):
    ewh = eh // NW
    nch = ewh // CS
    assert ewh % 8 == 0 and nch % NBUF == 0

    @functools.partial(
        pl.kernel,
        out_type=jax.ShapeDtypeStruct((NC * NP, D), jnp.float32),
        mesh=_mesh,
        scratch_types=[
            pltpu.VMEM((NBUF, CS), jnp.int32),
            pltpu.VMEM((NBUF, CS, D), jnp.float32),
            pltpu.VMEM_SHARED((NP, D), jnp.float32),
            pltpu.SemaphoreType.DMA((NBUF,)),
            pltpu.SemaphoreType.DMA((NBUF,)),
            pltpu.SemaphoreType.DMA((NBUF,)),
        ],
    )
    def seg_sum(edge_hbm, rcv_hbm, zeros_hbm, out_hbm, idx, bufs, acc,
                isem, lsem, ssem):
        c = lax.axis_index("c")
        s = lax.axis_index("s")
        w = s * NC + c
        base0 = w * ewh
        pltpu.sync_copy(zeros_hbm, acc.at[pl.ds(s * STRIPE, STRIPE), :])
        plsc.subcore_barrier()

        def ld(j, b):
            pltpu.async_copy(rcv_hbm.at[pl.ds(base0 + j * CS, CS)], idx.at[b],
                             isem.at[b])
            pltpu.async_copy(edge_hbm.at[pl.ds(base0 + j * CS, CS), :],
                             bufs.at[b], lsem.at[b])

        def sadd(j, b):
            pltpu.make_async_copy(rcv_hbm.at[pl.ds(base0 + j * CS, CS)],
                                  idx.at[b], isem.at[b]).wait()
            pltpu.make_async_copy(edge_hbm.at[pl.ds(base0 + j * CS, CS), :],
                                  bufs.at[b], lsem.at[b]).wait()
            pltpu.async_copy(bufs.at[b], acc.at[idx.at[b]], ssem.at[b],
                             add=True)

        def sadd_wait(b):
            pltpu.make_async_copy(bufs.at[b], acc.at[idx.at[b]],
                                  ssem.at[b]).wait()

        def outer(i, carry):
            for b in range(NBUF):
                j = i * NBUF + b

                @pl.when(j >= 1)
                def _():
                    sadd(j - 1, (b - 1) % NBUF)

                @pl.when(j >= NBUF)
                def _():
                    sadd_wait(b)

                ld(j, b)
            return carry

        lax.fori_loop(0, nch // NBUF, outer, 0)
        sadd(nch - 1, (nch - 1) % NBUF)
        for b in range(NBUF):
            sadd_wait(b)
        plsc.subcore_barrier()
        pltpu.sync_copy(
            acc.at[pl.ds(s * STRIPE, STRIPE), :],
            out_hbm.at[pl.ds(c * NP + s * STRIPE, STRIPE), :],
        )

    return seg_sum


_gathers = [_make_gather(eh) for eh in SPLITS]
_seg_sums = [_make_seg_sum(eh) for eh in SPLITS]


BN = 2000
BEDGE = 2000


def _proj_body(nf, w1s, w1r, ps, pr):
    x = nf[...]
    ps[...] = jnp.dot(x, w1s[...], preferred_element_type=jnp.float32)
    pr[...] = jnp.dot(x, w1r[...], preferred_element_type=jnp.float32)


_proj = pl.pallas_call(
    _proj_body,
    grid=(N // BN,),
    in_specs=[
        pl.BlockSpec((BN, D), lambda i: (i, 0)),
        pl.BlockSpec((D, D), lambda i: (0, 0)),
        pl.BlockSpec((D, D), lambda i: (0, 0)),
    ],
    out_specs=[pl.BlockSpec((BN, D), lambda i: (i, 0))] * 2,
    out_shape=[jax.ShapeDtypeStruct((N, D), jnp.float32)] * 2,
)


def _layer_norm(h, g, beta):
    mu = jnp.mean(h, axis=-1, keepdims=True)
    d = h - mu
    var = jnp.mean(d * d, axis=-1, keepdims=True)
    return d * lax.rsqrt(var + 1e-5) * g + beta


def _edge_mlp_body0(gath, ef, w1e, b1, w2, b2, g, beta, new_edge, out_edge):
    e = ef[...]
    h = gath[...] + jnp.dot(e, w1e[...], preferred_element_type=jnp.float32)
    h = jnp.maximum(h + b1[...], 0.0)
    h = jnp.dot(h, w2[...], preferred_element_type=jnp.float32) + b2[...]
    y = _layer_norm(h, g[...], beta[...])
    new_edge[...] = y
    out_edge[...] = y + e


def _edge_mlp_body1(gath, ef, w1e, b1, w2, b2, g, beta, oe_prev,
                    new_edge, out_edge):
    del oe_prev
    _edge_mlp_body0(gath, ef, w1e, b1, w2, b2, g, beta, new_edge, out_edge)


_W_SPECS = [
    pl.BlockSpec((D, D), lambda i: (0, 0)),
    pl.BlockSpec((1, D), lambda i: (0, 0)),
    pl.BlockSpec((D, D), lambda i: (0, 0)),
    pl.BlockSpec((1, D), lambda i: (0, 0)),
    pl.BlockSpec((1, D), lambda i: (0, 0)),
    pl.BlockSpec((1, D), lambda i: (0, 0)),
]


def _make_edge_mlp(k):
    nbe = SPLITS[k] // BEDGE
    off = OFFS[k] // BEDGE
    body = _edge_mlp_body0 if k == 0 else _edge_mlp_body1
    in_specs = [
        pl.BlockSpec((BEDGE, D), lambda i: (i, 0)),
        pl.BlockSpec((BEDGE, D), lambda i: (i + off, 0)),
    ] + _W_SPECS
    kwargs = {}
    if k > 0:
        in_specs = in_specs + [pl.BlockSpec(memory_space=pl.MemorySpace.ANY)]
        kwargs["input_output_aliases"] = {8: 1}
    return pl.pallas_call(
        body,
        grid=(nbe,),
        in_specs=in_specs,
        out_specs=[
            pl.BlockSpec((BEDGE, D), lambda i: (i, 0)),
            pl.BlockSpec((BEDGE, D), lambda i: (i + off, 0)),
        ],
        out_shape=[
            jax.ShapeDtypeStruct((SPLITS[k], D), jnp.float32),
            jax.ShapeDtypeStruct((E, D), jnp.float32),
        ],
        **kwargs,
    )


_edge_mlps = [_make_edge_mlp(k) for k in range(len(SPLITS))]

_NPARTS = 2 * len(SPLITS)


def _node_mlp_body(nf, *rest):
    parts = rest[:_NPARTS]
    w1a, w1b, b1, w2, b2, g, beta, out = rest[_NPARTS:]
    x = nf[...]
    seg = parts[0][...]
    for p in parts[1:]:
        seg = seg + p[...]
    h = jnp.dot(x, w1a[...], preferred_element_type=jnp.float32)
    h = h + jnp.dot(seg, w1b[...], preferred_element_type=jnp.float32)
    h = jnp.maximum(h + b1[...], 0.0)
    h = jnp.dot(h, w2[...], preferred_element_type=jnp.float32) + b2[...]
    out[...] = _layer_norm(h, g[...], beta[...]) + x


_node_mlp = pl.pallas_call(
    _node_mlp_body,
    grid=(N // BN,),
    in_specs=[pl.BlockSpec((BN, D), lambda i: (i, 0))] * (1 + _NPARTS) + [
        pl.BlockSpec((D, D), lambda i: (0, 0)),
        pl.BlockSpec((D, D), lambda i: (0, 0)),
        pl.BlockSpec((1, D), lambda i: (0, 0)),
        pl.BlockSpec((D, D), lambda i: (0, 0)),
        pl.BlockSpec((1, D), lambda i: (0, 0)),
        pl.BlockSpec((1, D), lambda i: (0, 0)),
        pl.BlockSpec((1, D), lambda i: (0, 0)),
    ],
    out_specs=pl.BlockSpec((BN, D), lambda i: (i, 0)),
    out_shape=jax.ShapeDtypeStruct((N, D), jnp.float32),
)


def kernel(node_features, edge_features, senders, receivers,
           edge_w1, edge_b1, edge_w2, edge_b2, edge_g, edge_beta,
           node_w1, node_b1, node_w2, node_b2, node_g, node_beta):
    ps, pr = _proj(node_features, edge_w1[:D], edge_w1[D:2 * D])
    w1e = edge_w1[2 * D:]
    eb1 = edge_b1.reshape(1, D)
    eb2 = edge_b2.reshape(1, D)
    eg = edge_g.reshape(1, D)
    ebt = edge_beta.reshape(1, D)
    zeros = jnp.zeros((STRIPE, D), jnp.float32)

    oe = None
    parts = []
    gaths = []
    rcvs = []
    for k, (eh, off) in enumerate(zip(SPLITS, OFFS)):
        ewh = eh // NW
        snd_k = lax.dynamic_slice_in_dim(senders, off, eh).reshape(
            NW, ewh // CG, CG)
        rcv_k = lax.dynamic_slice_in_dim(receivers, off, eh)
        rcvs.append(rcv_k)
        gaths.append(_gathers[k](ps, pr, snd_k,
                                 rcv_k.reshape(NW, ewh // CG, CG)))
    for k in range(len(SPLITS)):
        args = (gaths[k], edge_features, w1e, eb1, edge_w2, eb2, eg, ebt)
        if k > 0:
            args = args + (oe,)
        ne_k, oe = _edge_mlps[k](*args)
        parts.append(_seg_sums[k](ne_k, rcvs[k], zeros))

    pslices = []
    for p in parts:
        pslices += [p[:N], p[NP:NP + N]]
    out_node = _node_mlp(
        node_features, *pslices, node_w1[:D], node_w1[D:],
        node_b1.reshape(1, D), node_w2, node_b2.reshape(1, D),
        node_g.reshape(1, D), node_beta.reshape(1, D))
    return (out_node, oe)


# speedup vs baseline: 1.0719x; 1.0719x over previous
import functools

import jax
import jax.numpy as jnp
from jax import lax
from jax.experimental import pallas as pl
from jax.experimental.pallas import tpu as pltpu
from jax.experimental.pallas import tpu_sc as plsc

N = 10000
E = 320000
D = 128
NC = 2
NS = 16
NW = NC * NS
SPLITS = (128000, 192000)
OFFS = (0, 128000)
CG = 80
CS = 40
NBUF = 5
STRIPE = 632
NP = NS * STRIPE

_mesh = plsc.VectorSubcoreMesh(core_axis_name="c", subcore_axis_name="s")


def _make_gather(eh):
    ewh = eh // NW
    nch = ewh // CG
    assert ewh % 8 == 0 and nch % NBUF == 0

    @functools.partial(
        pl.kernel,
        out_type=jax.ShapeDtypeStruct((eh, D), jnp.float32),
        mesh=_mesh,
        scratch_types=[
            pltpu.VMEM((nch, CG), jnp.int32),
            pltpu.VMEM((nch, CG), jnp.int32),
            pltpu.VMEM((NBUF, CG, D), jnp.float32),
            pltpu.SemaphoreType.DMA((NBUF,)),
            pltpu.SemaphoreType.DMA((NBUF,)),
            pltpu.SemaphoreType.DMA((NBUF,)),
        ],
    )
    def gather_sum(ps_hbm, pr_hbm, snd_hbm, rcv_hbm, out_hbm,
                   idx_s, idx_r, bufs, gsem, asem, wsem):
        w = lax.axis_index("s") * NC + lax.axis_index("c")
        base0 = w * ewh
        pltpu.sync_copy(snd_hbm.at[w], idx_s)
        pltpu.sync_copy(rcv_hbm.at[w], idx_r)

        def gs(j, b):
            pltpu.async_copy(ps_hbm.at[idx_s.at[j]], bufs.at[b], gsem.at[b])

        def ga(j, b):
            pltpu.make_async_copy(ps_hbm.at[idx_s.at[j]], bufs.at[b],
                                  gsem.at[b]).wait()
            pltpu.async_copy(pr_hbm.at[idx_r.at[j]], bufs.at[b], asem.at[b],
                             add=True)

        def wr(j, b):
            pltpu.make_async_copy(pr_hbm.at[idx_r.at[j]], bufs.at[b],
                                  asem.at[b]).wait()
            pltpu.async_copy(bufs.at[b],
                             out_hbm.at[pl.ds(base0 + j * CG, CG), :],
                             wsem.at[b])

        def wr_wait(j, b):
            pltpu.make_async_copy(bufs.at[b],
                                  out_hbm.at[pl.ds(base0 + j * CG, CG), :],
                                  wsem.at[b]).wait()

        def outer(i, carry):
            for b in range(NBUF):
                j = i * NBUF + b

                @pl.when(j >= 2)
                def _():
                    wr(j - 2, (b - 2) % NBUF)

                @pl.when(j >= 1)
                def _():
                    ga(j - 1, (b - 1) % NBUF)

                @pl.when(j >= NBUF)
                def _():
                    wr_wait(j - NBUF, b)

                gs(j, b)
            return carry

        lax.fori_loop(0, nch // NBUF, outer, 0)
        ga(nch - 1, (nch - 1) % NBUF)
        wr(nch - 2, (nch - 2) % NBUF)
        wr(nch - 1, (nch - 1) % NBUF)
        for b in range(NBUF):
            wr_wait(nch - NBUF + b, b)

    return gather_sum


def _make_seg_sum(eh, cs=CS, nbuf=NBUF):
    ewh = eh // NW
    nch = ewh // cs
    assert ewh % 8 == 0 and nch % nbuf == 0

    @functools.partial(
        pl.kernel,
        out_type=jax.ShapeDtypeStruct((NC * NP, D), jnp.float32),
        mesh=_mesh,
        scratch_types=[
            pltpu.VMEM((nbuf, cs), jnp.int32),
            pltpu.VMEM((nbuf, cs, D), jnp.float32),
            pltpu.VMEM_SHARED((NP, D), jnp.float32),
            pltpu.SemaphoreType.DMA((nbuf,)),
            pltpu.SemaphoreType.DMA((nbuf,)),
            pltpu.SemaphoreType.DMA((nbuf,)),
        ],
    )
    def seg_sum(edge_hbm, rcv_hbm, zeros_hbm, out_hbm, idx, bufs, acc,
                isem, lsem, ssem):
        c = lax.axis_index("c")
        s = lax.axis_index("s")
        w = s * NC + c
        base0 = w * ewh
        pltpu.sync_copy(zeros_hbm, acc.at[pl.ds(s * STRIPE, STRIPE), :])
        plsc.subcore_barrier()

        def ld(j, b):
            pltpu.async_copy(rcv_hbm.at[pl.ds(base0 + j * cs, cs)], idx.at[b],
                             isem.at[b])
            pltpu.async_copy(edge_hbm.at[pl.ds(base0 + j * cs, cs), :],
                             bufs.at[b], lsem.at[b])

        def sadd(j, b):
            pltpu.make_async_copy(rcv_hbm.at[pl.ds(base0 + j * cs, cs)],
                                  idx.at[b], isem.at[b]).wait()
            pltpu.make_async_copy(edge_hbm.at[pl.ds(base0 + j * cs, cs), :],
                                  bufs.at[b], lsem.at[b]).wait()
            pltpu.async_copy(bufs.at[b], acc.at[idx.at[b]], ssem.at[b],
                             add=True)

        def sadd_wait(b):
            pltpu.make_async_copy(bufs.at[b], acc.at[idx.at[b]],
                                  ssem.at[b]).wait()

        def outer(i, carry):
            for b in range(nbuf):
                j = i * nbuf + b

                @pl.when(j >= 1)
                def _():
                    sadd(j - 1, (b - 1) % nbuf)

                @pl.when(j >= nbuf)
                def _():
                    sadd_wait(b)

                ld(j, b)
            return carry

        lax.fori_loop(0, nch // nbuf, outer, 0)
        sadd(nch - 1, (nch - 1) % nbuf)
        for b in range(nbuf):
            sadd_wait(b)
        plsc.subcore_barrier()
        pltpu.sync_copy(
            acc.at[pl.ds(s * STRIPE, STRIPE), :],
            out_hbm.at[pl.ds(c * NP + s * STRIPE, STRIPE), :],
        )

    return seg_sum


_gathers = [_make_gather(eh) for eh in SPLITS]
_seg_sums = [_make_seg_sum(SPLITS[0], 40, 5), _make_seg_sum(SPLITS[1], 80, 3)]


BN = 2000
BEDGE = 2000


def _proj_body(nf, w1s, w1r, ps, pr):
    x = nf[...]
    ps[...] = jnp.dot(x, w1s[...], preferred_element_type=jnp.float32)
    pr[...] = jnp.dot(x, w1r[...], preferred_element_type=jnp.float32)


_proj = pl.pallas_call(
    _proj_body,
    grid=(N // BN,),
    in_specs=[
        pl.BlockSpec((BN, D), lambda i: (i, 0)),
        pl.BlockSpec((D, D), lambda i: (0, 0)),
        pl.BlockSpec((D, D), lambda i: (0, 0)),
    ],
    out_specs=[pl.BlockSpec((BN, D), lambda i: (i, 0))] * 2,
    out_shape=[jax.ShapeDtypeStruct((N, D), jnp.float32)] * 2,
)


def _layer_norm(h, g, beta):
    mu = jnp.mean(h, axis=-1, keepdims=True)
    d = h - mu
    var = jnp.mean(d * d, axis=-1, keepdims=True)
    return d * lax.rsqrt(var + 1e-5) * g + beta


def _edge_mlp_body0(gath, ef, w1e, b1, w2, b2, g, beta, new_edge, out_edge):
    e = ef[...]
    h = gath[...] + jnp.dot(e, w1e[...], preferred_element_type=jnp.float32)
    h = jnp.maximum(h + b1[...], 0.0)
    h = jnp.dot(h, w2[...], preferred_element_type=jnp.float32) + b2[...]
    y = _layer_norm(h, g[...], beta[...])
    new_edge[...] = y
    out_edge[...] = y + e


def _edge_mlp_body1(gath, ef, w1e, b1, w2, b2, g, beta, oe_prev,
                    new_edge, out_edge):
    del oe_prev
    _edge_mlp_body0(gath, ef, w1e, b1, w2, b2, g, beta, new_edge, out_edge)


_W_SPECS = [
    pl.BlockSpec((D, D), lambda i: (0, 0)),
    pl.BlockSpec((1, D), lambda i: (0, 0)),
    pl.BlockSpec((D, D), lambda i: (0, 0)),
    pl.BlockSpec((1, D), lambda i: (0, 0)),
    pl.BlockSpec((1, D), lambda i: (0, 0)),
    pl.BlockSpec((1, D), lambda i: (0, 0)),
]


def _make_edge_mlp(k):
    nbe = SPLITS[k] // BEDGE
    off = OFFS[k] // BEDGE
    body = _edge_mlp_body0 if k == 0 else _edge_mlp_body1
    in_specs = [
        pl.BlockSpec((BEDGE, D), lambda i: (i, 0)),
        pl.BlockSpec((BEDGE, D), lambda i: (i + off, 0)),
    ] + _W_SPECS
    kwargs = {}
    if k > 0:
        in_specs = in_specs + [pl.BlockSpec(memory_space=pl.MemorySpace.ANY)]
        kwargs["input_output_aliases"] = {8: 1}
    return pl.pallas_call(
        body,
        grid=(nbe,),
        in_specs=in_specs,
        out_specs=[
            pl.BlockSpec((BEDGE, D), lambda i: (i, 0)),
            pl.BlockSpec((BEDGE, D), lambda i: (i + off, 0)),
        ],
        out_shape=[
            jax.ShapeDtypeStruct((SPLITS[k], D), jnp.float32),
            jax.ShapeDtypeStruct((E, D), jnp.float32),
        ],
        **kwargs,
    )


_edge_mlps = [_make_edge_mlp(k) for k in range(len(SPLITS))]

_NPARTS = 2 * len(SPLITS)


def _node_mlp_body(nf, *rest):
    parts = rest[:_NPARTS]
    w1a, w1b, b1, w2, b2, g, beta, out = rest[_NPARTS:]
    x = nf[...]
    seg = parts[0][...]
    for p in parts[1:]:
        seg = seg + p[...]
    h = jnp.dot(x, w1a[...], preferred_element_type=jnp.float32)
    h = h + jnp.dot(seg, w1b[...], preferred_element_type=jnp.float32)
    h = jnp.maximum(h + b1[...], 0.0)
    h = jnp.dot(h, w2[...], preferred_element_type=jnp.float32) + b2[...]
    out[...] = _layer_norm(h, g[...], beta[...]) + x


_node_mlp = pl.pallas_call(
    _node_mlp_body,
    grid=(N // BN,),
    in_specs=[pl.BlockSpec((BN, D), lambda i: (i, 0))] * (1 + _NPARTS) + [
        pl.BlockSpec((D, D), lambda i: (0, 0)),
        pl.BlockSpec((D, D), lambda i: (0, 0)),
        pl.BlockSpec((1, D), lambda i: (0, 0)),
        pl.BlockSpec((D, D), lambda i: (0, 0)),
        pl.BlockSpec((1, D), lambda i: (0, 0)),
        pl.BlockSpec((1, D), lambda i: (0, 0)),
        pl.BlockSpec((1, D), lambda i: (0, 0)),
    ],
    out_specs=pl.BlockSpec((BN, D), lambda i: (i, 0)),
    out_shape=jax.ShapeDtypeStruct((N, D), jnp.float32),
)


def kernel(node_features, edge_features, senders, receivers,
           edge_w1, edge_b1, edge_w2, edge_b2, edge_g, edge_beta,
           node_w1, node_b1, node_w2, node_b2, node_g, node_beta):
    ps, pr = _proj(node_features, edge_w1[:D], edge_w1[D:2 * D])
    w1e = edge_w1[2 * D:]
    eb1 = edge_b1.reshape(1, D)
    eb2 = edge_b2.reshape(1, D)
    eg = edge_g.reshape(1, D)
    ebt = edge_beta.reshape(1, D)
    zeros = jnp.zeros((STRIPE, D), jnp.float32)

    oe = None
    parts = []
    gaths = []
    rcvs = []
    for k, (eh, off) in enumerate(zip(SPLITS, OFFS)):
        ewh = eh // NW
        snd_k = lax.dynamic_slice_in_dim(senders, off, eh).reshape(
            NW, ewh // CG, CG)
        rcv_k = lax.dynamic_slice_in_dim(receivers, off, eh)
        rcvs.append(rcv_k)
        gaths.append(_gathers[k](ps, pr, snd_k,
                                 rcv_k.reshape(NW, ewh // CG, CG)))
    for k in range(len(SPLITS)):
        args = (gaths[k], edge_features, w1e, eb1, edge_w2, eb2, eg, ebt)
        if k > 0:
            args = args + (oe,)
        ne_k, oe = _edge_mlps[k](*args)
        parts.append(_seg_sums[k](ne_k, rcvs[k], zeros))

    pslices = []
    for p in parts:
        pslices += [p[:N], p[NP:NP + N]]
    out_node = _node_mlp(
        node_features, *pslices, node_w1[:D], node_w1[D:],
        node_b1.reshape(1, D), node_w2, node_b2.reshape(1, D),
        node_g.reshape(1, D), node_beta.reshape(1, D))
    return (out_node, oe)
